# baseline (speedup 1.0000x reference)
import functools

import jax
import jax.numpy as jnp
from jax import lax
from jax.experimental import pallas as pl
from jax.experimental.pallas import tpu as pltpu

B = 16384
D = 128
K = 64
M = 8
KM = K * M
BLK = 2048


def _main_kernel(cls_ref, wq_ref, bq_ref, g_ref, b_ref,
                 cp_ref, ff_ref, wp_ref, bp_ref,
                 prompt_ref, cwT_ref, fwT_ref,
                 cn_s, fn_s, fp_s):
    @pl.when(pl.program_id(0) == 0)
    def _prep():
        cp = cp_ref[:]
        cn_s[:] = cp / jnp.maximum(
            jnp.sqrt(jnp.sum(cp * cp, axis=1, keepdims=True)), 1e-12)
        ff = ff_ref[:]
        fn_s[:] = ff / jnp.maximum(
            jnp.sqrt(jnp.sum(ff * ff, axis=1, keepdims=True)), 1e-12)
        fp_s[:] = lax.dot_general(
            ff, wp_ref[:], (((1,), (1,)), ((), ())),
            preferred_element_type=jnp.float32) + bp_ref[:]

    qT = lax.dot_general(wq_ref[:], cls_ref[:], (((1,), (1,)), ((), ())),
                         preferred_element_type=jnp.float32) + bq_ref[:]
    mu = jnp.mean(qT, axis=0, keepdims=True)
    c = qT - mu
    var = jnp.mean(c * c, axis=0, keepdims=True)
    qn = c / jnp.sqrt(var + 1e-5) * g_ref[:] + b_ref[:]
    nrm = jnp.maximum(jnp.sqrt(jnp.sum(qn * qn, axis=0, keepdims=True)), 1e-12)
    q1T = qn / nrm

    csT = lax.dot_general(cn_s[:], q1T, (((1,), (0,)), ((), ())),
                          preferred_element_type=jnp.float32)
    ecT = jnp.exp(2.0 * csT)
    cwT = ecT / jnp.sum(ecT, axis=0, keepdims=True)

    wi = cwT[None, :, :]
    shiftedT = jnp.zeros_like(cwT)
    for jc in range(0, K, 8):
        wj = cwT[jc:jc + 8][:, None, :]
        shiftedT = shiftedT + jnp.sum(
            jnp.where(wj > wi, wj, 0.0), axis=0)
    maskT = (shiftedT <= 0.9).astype(jnp.float32)

    fsT = lax.dot_general(fn_s[:], q1T, (((1,), (0,)), ((), ())),
                          preferred_element_type=jnp.float32)
    efT = jnp.exp(2.0 * fsT)
    grp = (lax.broadcasted_iota(jnp.int32, (KM, K), 0) // M ==
           lax.broadcasted_iota(jnp.int32, (KM, K), 1)).astype(jnp.float32)
    gsT = lax.dot_general(grp, efT, (((0,), (0,)), ((), ())),
                          preferred_element_type=jnp.float32)
    t1 = maskT / gsT
    t2 = cwT * t1
    rep1 = lax.dot_general(grp, t1, (((1,), (0,)), ((), ())),
                           preferred_element_type=jnp.float32)
    rep2 = lax.dot_general(grp, t2, (((1,), (0,)), ((), ())),
                           preferred_element_type=jnp.float32)
    fwT_ref[:] = efT * rep1
    combT = efT * rep2
    prompt_ref[:] = lax.dot_general(combT, fp_s[:], (((0,), (0,)), ((), ())),
                                    preferred_element_type=jnp.float32)
    cwT_ref[:] = cwT


@functools.partial(jax.jit, static_argnames=())
def kernel(cls_token, coarse_prototypes, fine_prototypes, Wq, bq, Wp, bp,
           ln_g, ln_b):
    f32 = jnp.float32
    ff = fine_prototypes.reshape(KM, D)
    nblk = B // BLK
    row = lambda i: (i, 0)
    col = lambda i: (0, i)
    rep = lambda i: (0, 0)
    prompt, cwT, fwT = pl.pallas_call(
        _main_kernel,
        grid=(nblk,),
        in_specs=[
            pl.BlockSpec((BLK, D), row),
            pl.BlockSpec((D, D), rep),
            pl.BlockSpec((D, 1), rep),
            pl.BlockSpec((D, 1), rep),
            pl.BlockSpec((D, 1), rep),
            pl.BlockSpec((K, D), rep),
            pl.BlockSpec((KM, D), rep),
            pl.BlockSpec((D, D), rep),
            pl.BlockSpec((1, D), rep),
        ],
        out_specs=(
            pl.BlockSpec((BLK, D), row),
            pl.BlockSpec((K, BLK), col),
            pl.BlockSpec((KM, BLK), col),
        ),
        out_shape=(
            jax.ShapeDtypeStruct((B, D), f32),
            jax.ShapeDtypeStruct((K, B), f32),
            jax.ShapeDtypeStruct((KM, B), f32),
        ),
        scratch_shapes=[
            pltpu.VMEM((K, D), f32),
            pltpu.VMEM((KM, D), f32),
            pltpu.VMEM((KM, D), f32),
        ],
        compiler_params=pltpu.CompilerParams(
            dimension_semantics=("arbitrary",)),
    )(cls_token, Wq, bq.reshape(D, 1), ln_g.reshape(D, 1),
      ln_b.reshape(D, 1), coarse_prototypes, ff, Wp, bp.reshape(1, D))
    return prompt, cwT.T, fwT.T.reshape(B, K, M)
